# Initial kernel scaffold; baseline (speedup 1.0000x reference)
import jax
import jax.numpy as jnp
from jax.experimental import pallas as pl


def kernel(node_feats, edge_index, edge_attr, coords, batch_node_vec, ew1s, eb1s, ew2s, eb2s, nw1s, nb1s, nw2s, nb2s, cw1s, cb1s, cw2s, cmws, cmbs, post_w, post_b):
    raise NotImplementedError("write your pallas kernel here")


# speedup vs baseline: 2.5035x; 2.5035x over previous
import functools

import jax
import jax.numpy as jnp
from jax import lax
from jax.experimental import pallas as pl
from jax.experimental.pallas import tpu as pltpu
from jax.experimental.pallas import tpu_sc as plsc

N_NODES = 10000
N_EDGES = 320000
D_FEAT = 128
D_EDGE = 16
H = 128
N_GRAPHS = 64
N_LAYERS = 3

NC = 2
NS = 16
NW = NC * NS
EPW = N_EDGES // NW
CH = 80
NCH = EPW // CH
NPAD = 10240
NSH = 8
NPART = 4
PROWS = NPAD // NPART
CHS = 256
ESH = 40960
EPADS = NSH * ESH


@functools.cache
def _sc_kernels():
    mesh = plsc.VectorSubcoreMesh(core_axis_name="c", subcore_axis_name="s",
                                  num_cores=NC, num_subcores=NS)

    @functools.partial(
        pl.kernel,
        out_type=(
            jax.ShapeDtypeStruct((N_EDGES, D_FEAT), jnp.float32),
            jax.ShapeDtypeStruct((N_EDGES, 16), jnp.float32),
        ),
        mesh=mesh,
        compiler_params=pltpu.CompilerParams(needs_layout_passes=False),
        scratch_types=[
            pltpu.VMEM((CH,), jnp.int32),
            pltpu.VMEM((CH,), jnp.int32),
            pltpu.VMEM((CH, D_FEAT), jnp.float32),
            pltpu.VMEM((CH, D_FEAT), jnp.float32),
            pltpu.VMEM((CH, 16), jnp.float32),
            pltpu.VMEM((N_NODES // 2000, 8, 2000), jnp.float32),
            pltpu.SemaphoreType.DMA,
        ],
    )
    def _sc_gather(p_hbm, q_hbm, ct_hbm, src_hbm, dst_hbm,
                   s_out, d_out, idx_s, idx_d, buf_p, buf_q, buf_c, xyz, sem):
        wid = lax.axis_index("s") * NC + lax.axis_index("c")
        base = wid * EPW

        pltpu.sync_copy(ct_hbm, xyz)

        def zrow(j, carry):
            buf_c[j, pl.ds(0, 16)] = jnp.zeros((16,), jnp.float32)
            return carry

        lax.fori_loop(0, CH, zrow, 0, unroll=4)

        def chunk(i, carry):
            off = base + i * CH
            pltpu.sync_copy(src_hbm.at[pl.ds(off, CH)], idx_s)
            pltpu.sync_copy(dst_hbm.at[pl.ds(off, CH)], idx_d)
            cp1 = pltpu.async_copy(p_hbm.at[idx_s], buf_p, sem)
            cp2 = pltpu.async_copy(q_hbm.at[idx_d], buf_q, sem)
            cp1.wait()
            cp2.wait()

            def group(g, c2):
                s16 = idx_s[pl.ds(g * 16, 16)]
                d16 = idx_d[pl.ds(g * 16, 16)]
                sb = s16 // 2000
                sc = s16 - sb * 2000
                db = d16 // 2000
                dc_ = d16 - db * 2000
                rows = lax.iota(jnp.int32, 16) + g * 16
                for c in range(3):
                    csplat = jnp.full((16,), c, jnp.int32)
                    dcv = (plsc.load_gather(xyz, [sb, csplat, sc])
                           - plsc.load_gather(xyz, [db, csplat, dc_]))
                    plsc.store_scatter(buf_c, [rows, jnp.full((16,), c,
                                                              jnp.int32)], dcv)
                return c2

            def edge(j, c2):
                for k in range(D_FEAT // 16):
                    plsc.addupdate(buf_p.at[j, pl.ds(k * 16, 16)],
                                   buf_q[j, pl.ds(k * 16, 16)])
                return c2

            lax.fori_loop(0, CH // 16, group, 0)
            lax.fori_loop(0, CH, edge, 0, unroll=2)
            pltpu.sync_copy(buf_p, s_out.at[pl.ds(off, CH)])
            pltpu.sync_copy(buf_c, d_out.at[pl.ds(off, CH)])
            return carry

        lax.fori_loop(0, NCH, chunk, 0)

    @functools.partial(
        pl.kernel,
        out_type=jax.ShapeDtypeStruct((NSH * NPAD * 32,), jnp.float32),
        mesh=mesh,
        compiler_params=pltpu.CompilerParams(needs_layout_passes=False),
        scratch_types=[
            pltpu.VMEM((CHS,), jnp.int32),
            pltpu.VMEM((32, CHS), jnp.float32),
            pltpu.VMEM((PROWS * 32,), jnp.float32),
        ],
    )
    def _sc_scatter(ett_hbm, src_hbm, agg_out, idx_v, buf_t, tflat):
        wid = lax.axis_index("s") * NC + lax.axis_index("c")
        share = wid // NPART
        part = wid - share * NPART
        ebase = share * ESH
        lo = part * PROWS

        zvec = jnp.zeros((16,), jnp.float32)

        def zrow(i, carry):
            tflat[pl.ds(i * 16, 16)] = zvec
            return carry

        lax.fori_loop(0, PROWS * 2, zrow, 0, unroll=4)

        def chunk(i, carry):
            off = ebase + i * CHS
            pltpu.sync_copy(src_hbm.at[pl.ds(off, CHS)], idx_v)
            pltpu.sync_copy(ett_hbm.at[:, pl.ds(off, CHS)], buf_t)

            def group(g, c2):
                s16 = idx_v[pl.ds(g * 16, 16)]
                tgt = s16 - lo
                mask = (tgt >= 0) & (tgt < PROWS)
                tgt = jnp.minimum(jnp.maximum(tgt, 0), PROWS - 1)
                fidx = tgt * 32
                for c in range(32):
                    vec = buf_t[c, pl.ds(g * 16, 16)]
                    plsc.addupdate_scatter(tflat, [fidx + c], vec, mask=mask)
                return c2

            lax.fori_loop(0, CHS // 16, group, 0)
            return carry

        lax.fori_loop(0, ESH // CHS, chunk, 0)
        pltpu.sync_copy(
            tflat,
            agg_out.at[pl.ds(share * (NPAD * 32) + part * (PROWS * 32),
                             PROWS * 32)])

    return _sc_gather, _sc_scatter


_NT = (((1,), (1,)), ((), ()))


def _init_body(nf_ref, c16_ref, ws_ref, wd_ref, eye_ref, p_ref, q_ref,
               ct_ref):
    nf = nf_ref[...]
    p_ref[...] = jnp.dot(nf, ws_ref[...], preferred_element_type=jnp.float32)
    q_ref[...] = jnp.dot(nf, wd_ref[...], preferred_element_type=jnp.float32)
    ct_ref[0] = lax.dot_general(eye_ref[...], c16_ref[...], _NT,
                                preferred_element_type=jnp.float32)


def _edge_body(s_ref, d_ref, ea_ref, we_ref, b1_ref, wr_ref, w2_ref, b2_ref,
               cw1_ref, cb1_ref, cw2_ref, sel_ref, eye_ref, et_ref, *,
               ea_rows):
    S = s_ref[...]
    D = d_ref[...]
    if ea_rows:
        EA = lax.dot_general(ea_ref[...], sel_ref[...],
                             (((0,), (0,)), ((), ())),
                             preferred_element_type=jnp.float32)
    else:
        EA = ea_ref[...]
    radial = jnp.sum(D * D, axis=1, keepdims=True)
    pre = S + jnp.dot(EA, we_ref[...], preferred_element_type=jnp.float32)
    pre = pre + radial * wr_ref[...] + b1_ref[...]
    h = jnp.maximum(pre, 0.0)
    eo = jnp.dot(h, w2_ref[...], preferred_element_type=jnp.float32) + b2_ref[...]
    t0 = jnp.maximum(
        jnp.sum(eo * cw1_ref[0:1, :], axis=1, keepdims=True) + cb1_ref[0, 0], 0.0)
    t1 = jnp.maximum(
        jnp.sum(eo * cw1_ref[1:2, :], axis=1, keepdims=True) + cb1_ref[0, 1], 0.0)
    eps = t0 * cw2_ref[0, 0] + t1 * cw2_ref[0, 1]
    norm = jnp.sqrt(radial) + 1.0
    trans = (D / norm) * eps
    cat = jnp.concatenate([eo, trans], axis=1)
    et_ref[...] = lax.dot_general(eye_ref[...], cat, _NT,
                                  preferred_element_type=jnp.float32)


def _node_mid_body(nf_ref, agg_ref, c16_ref,
                   wcm_ref, wcmb_ref, wnf_ref, wne_ref, wnc_ref, nb1_ref,
                   wn2_ref, nb2_ref, wsn_ref, wdn_ref, eye_ref,
                   nf2_ref, p_ref, q_ref, c16o_ref, ct_ref):
    agg = jnp.sum(agg_ref[...], axis=0)
    agg_e = agg[:, 0:16]
    agg_c = agg[:, 16:32]
    acc = c16_ref[...] + agg_c
    cout = jnp.dot(acc, wcm_ref[...], preferred_element_type=jnp.float32)
    cout = cout + wcmb_ref[...]
    hn = jnp.dot(nf_ref[...], wnf_ref[...], preferred_element_type=jnp.float32)
    hn = hn + jnp.dot(agg_e, wne_ref[...], preferred_element_type=jnp.float32)
    hn = hn + jnp.dot(cout, wnc_ref[...], preferred_element_type=jnp.float32)
    hn = jnp.maximum(hn + nb1_ref[...], 0.0)
    nf2 = jnp.dot(hn, wn2_ref[...], preferred_element_type=jnp.float32) + nb2_ref[...]
    nf2_ref[...] = nf2
    p_ref[...] = jnp.dot(nf2, wsn_ref[...], preferred_element_type=jnp.float32)
    q_ref[...] = jnp.dot(nf2, wdn_ref[...], preferred_element_type=jnp.float32)
    c16o_ref[...] = cout
    ct_ref[0] = lax.dot_general(eye_ref[...], cout, _NT,
                                preferred_element_type=jnp.float32)


def _node_final_body(nf_ref, agg_ref, c16_ref, batch_ref,
                     wcm_ref, wcmb_ref, wnf_ref, wne_ref, wnc_ref, nb1_ref,
                     wn2_ref, nb2_ref, pw_ref, pb_ref,
                     emb_ref, c16o_ref, graph_ref, sums_ref, counts_ref,
                     *, n_steps, block_n):
    step = pl.program_id(0)
    agg = jnp.sum(agg_ref[...], axis=0)
    agg_e = agg[:, 0:16]
    agg_c = agg[:, 16:32]
    acc = c16_ref[...] + agg_c
    cout = jnp.dot(acc, wcm_ref[...], preferred_element_type=jnp.float32)
    cout = cout + wcmb_ref[...]
    hn = jnp.dot(nf_ref[...], wnf_ref[...], preferred_element_type=jnp.float32)
    hn = hn + jnp.dot(agg_e, wne_ref[...], preferred_element_type=jnp.float32)
    hn = hn + jnp.dot(cout, wnc_ref[...], preferred_element_type=jnp.float32)
    hn = jnp.maximum(hn + nb1_ref[...], 0.0)
    nf2 = jnp.dot(hn, wn2_ref[...], preferred_element_type=jnp.float32) + nb2_ref[...]
    embs = jnp.dot(nf2, pw_ref[...], preferred_element_type=jnp.float32) + pb_ref[...]
    emb_ref[...] = embs
    c16o_ref[...] = cout

    bvec = batch_ref[0]
    oh = (lax.broadcasted_iota(jnp.int32, (N_GRAPHS, block_n), 0)
          == bvec).astype(jnp.float32)
    s_step = jnp.dot(oh, embs, preferred_element_type=jnp.float32)
    c_step = jnp.sum(oh, axis=1, keepdims=True)

    @pl.when(step == 0)
    def _():
        sums_ref[...] = jnp.zeros_like(sums_ref)
        counts_ref[...] = jnp.zeros_like(counts_ref)

    sums_ref[...] += s_step
    counts_ref[...] += c_step

    @pl.when(step == n_steps - 1)
    def _():
        graph_ref[...] = sums_ref[...] / jnp.maximum(counts_ref[...], 1.0)


def _bspec(block, imap):
    return pl.BlockSpec(block, imap)


def _full(shape):
    return pl.BlockSpec(shape, lambda i: tuple(0 for _ in shape))


def _tc_init(nf, c16, ws0, wd0, eye8, block_n=2000):
    n_steps = N_NODES // block_n
    return pl.pallas_call(
        _init_body,
        grid=(n_steps,),
        in_specs=[
            _bspec((block_n, D_FEAT), lambda i: (i, 0)),
            _bspec((block_n, 16), lambda i: (i, 0)),
            _full((D_FEAT, D_FEAT)),
            _full((D_FEAT, D_FEAT)),
            _full((8, 16)),
        ],
        out_specs=[
            _bspec((block_n, D_FEAT), lambda i: (i, 0)),
            _bspec((block_n, D_FEAT), lambda i: (i, 0)),
            _bspec((1, 8, block_n), lambda i: (i, 0, 0)),
        ],
        out_shape=[
            jax.ShapeDtypeStruct((N_NODES, D_FEAT), jnp.float32),
            jax.ShapeDtypeStruct((N_NODES, D_FEAT), jnp.float32),
            jax.ShapeDtypeStruct((n_steps, 8, block_n), jnp.float32),
        ],
    )(nf, c16, ws0, wd0, eye8)


def _tc_edge(S, D, EA, we, b1, wr, w2, b2, cw1, cb1, cw2, sel, eye32,
             block_e=2560):
    n_steps = N_EDGES // block_e
    ea_rows = (EA.shape[0] == 32)
    ea_spec = (_bspec((32, block_e), lambda i: (0, i)) if ea_rows
               else _bspec((block_e, D_EDGE), lambda i: (i, 0)))
    body = functools.partial(_edge_body, ea_rows=ea_rows)
    return pl.pallas_call(
        body,
        grid=(n_steps,),
        in_specs=[
            _bspec((block_e, D_FEAT), lambda i: (i, 0)),
            _bspec((block_e, 16), lambda i: (i, 0)),
            ea_spec,
            _full((D_EDGE, H)),
            _full((1, H)),
            _full((1, H)),
            _full((H, D_EDGE)),
            _full((1, D_EDGE)),
            _full((2, D_EDGE)),
            _full((1, 2)),
            _full((1, 2)),
            _full((32, D_EDGE)),
            _full((32, 32)),
        ],
        out_specs=[_bspec((32, block_e), lambda i: (0, i))],
        out_shape=[jax.ShapeDtypeStruct((32, EPADS), jnp.float32)],
    )(S, D, EA, we, b1, wr, w2, b2, cw1, cb1, cw2, sel, eye32)[0]


def _tc_node_mid(nf, agg, c16, wcm, wcmb, wnf, wne, wnc, nb1, wn2, nb2,
                 wsn, wdn, eye8, block_n=2000):
    n_steps = N_NODES // block_n
    row = lambda i: (i, 0)
    return pl.pallas_call(
        _node_mid_body,
        grid=(n_steps,),
        in_specs=[
            _bspec((block_n, D_FEAT), row),
            _bspec((NSH, block_n, 32), lambda i: (0, i, 0)),
            _bspec((block_n, 16), row),
            _full((16, 16)),
            _full((1, 16)),
            _full((D_FEAT, H)),
            _full((16, H)),
            _full((16, H)),
            _full((1, H)),
            _full((H, D_FEAT)),
            _full((1, D_FEAT)),
            _full((D_FEAT, D_FEAT)),
            _full((D_FEAT, D_FEAT)),
            _full((8, 16)),
        ],
        out_specs=[
            _bspec((block_n, D_FEAT), row),
            _bspec((block_n, D_FEAT), row),
            _bspec((block_n, D_FEAT), row),
            _bspec((block_n, 16), row),
            _bspec((1, 8, block_n), lambda i: (i, 0, 0)),
        ],
        out_shape=[
            jax.ShapeDtypeStruct((N_NODES, D_FEAT), jnp.float32),
            jax.ShapeDtypeStruct((N_NODES, D_FEAT), jnp.float32),
            jax.ShapeDtypeStruct((N_NODES, D_FEAT), jnp.float32),
            jax.ShapeDtypeStruct((NPAD, 16), jnp.float32),
            jax.ShapeDtypeStruct((n_steps, 8, block_n), jnp.float32),
        ],
    )(nf, agg, c16, wcm, wcmb, wnf, wne, wnc, nb1, wn2, nb2, wsn, wdn,
      eye8)


def _tc_node_final(nf, agg, c16, batch3, wcm, wcmb, wnf, wne, wnc, nb1,
                   wn2, nb2, pw, pb, block_n=2000):
    n_steps = N_NODES // block_n
    row = lambda i: (i, 0)
    body = functools.partial(_node_final_body, n_steps=n_steps, block_n=block_n)
    return pl.pallas_call(
        body,
        grid=(n_steps,),
        in_specs=[
            _bspec((block_n, D_FEAT), row),
            _bspec((NSH, block_n, 32), lambda i: (0, i, 0)),
            _bspec((block_n, 16), row),
            _bspec((1, 1, block_n), lambda i: (i, 0, 0)),
            _full((16, 16)),
            _full((1, 16)),
            _full((D_FEAT, H)),
            _full((16, H)),
            _full((16, H)),
            _full((1, H)),
            _full((H, D_FEAT)),
            _full((1, D_FEAT)),
            _full((D_FEAT, D_FEAT)),
            _full((1, D_FEAT)),
        ],
        out_specs=[
            _bspec((block_n, D_FEAT), row),
            _bspec((block_n, 16), row),
            _full((N_GRAPHS, D_FEAT)),
        ],
        out_shape=[
            jax.ShapeDtypeStruct((N_NODES, D_FEAT), jnp.float32),
            jax.ShapeDtypeStruct((N_NODES, 16), jnp.float32),
            jax.ShapeDtypeStruct((N_GRAPHS, D_FEAT), jnp.float32),
        ],
        scratch_shapes=[
            pltpu.VMEM((N_GRAPHS, D_FEAT), jnp.float32),
            pltpu.VMEM((N_GRAPHS, 1), jnp.float32),
        ],
    )(nf, agg, c16, batch3, wcm, wcmb, wnf, wne, wnc, nb1, wn2, nb2,
      pw, pb)


def kernel(node_feats, edge_index, edge_attr, coords, batch_node_vec,
           ew1s, eb1s, ew2s, eb2s, nw1s, nb1s, nw2s, nb2s,
           cw1s, cb1s, cw2s, cmws, cmbs, post_w, post_b):
    f32 = jnp.float32
    src = edge_index[0]
    dst = edge_index[1]

    ws = [ew1s[l][:, 0:D_FEAT].T for l in range(N_LAYERS)]
    wd = [ew1s[l][:, D_FEAT:2 * D_FEAT].T for l in range(N_LAYERS)]
    we = [ew1s[l][:, 2 * D_FEAT:2 * D_FEAT + D_EDGE].T for l in range(N_LAYERS)]
    wr = [ew1s[l][:, 2 * D_FEAT + D_EDGE:].T for l in range(N_LAYERS)]
    b1 = [eb1s[l].reshape(1, H) for l in range(N_LAYERS)]
    w2 = [ew2s[l].T for l in range(N_LAYERS)]
    b2 = [eb2s[l].reshape(1, D_EDGE) for l in range(N_LAYERS)]
    cw1 = [cw1s[l] for l in range(N_LAYERS)]
    cb1 = [cb1s[l].reshape(1, 2) for l in range(N_LAYERS)]
    cw2 = [cw2s[l] for l in range(N_LAYERS)]
    wcm = [jnp.zeros((16, 16), f32).at[0:3, 0:3].set(cmws[l].T)
           for l in range(N_LAYERS)]
    wcmb = [jnp.zeros((1, 16), f32).at[0, 0:3].set(cmbs[l])
            for l in range(N_LAYERS)]
    wnf = [nw1s[l][:, 0:D_FEAT].T for l in range(N_LAYERS)]
    wne = [nw1s[l][:, D_FEAT:D_FEAT + D_EDGE].T for l in range(N_LAYERS)]
    wnc = [jnp.zeros((16, H), f32).at[0:3, :].set(nw1s[l][:, D_FEAT + D_EDGE:].T)
           for l in range(N_LAYERS)]
    nb1 = [nb1s[l].reshape(1, H) for l in range(N_LAYERS)]
    wn2 = [nw2s[l].T for l in range(N_LAYERS)]
    nb2 = [nb2s[l].reshape(1, D_FEAT) for l in range(N_LAYERS)]
    pw = post_w.T
    pb = post_b.reshape(1, D_FEAT)

    c16 = jnp.pad(coords, ((0, NPAD - N_NODES), (0, 13)))
    batch3 = batch_node_vec.astype(jnp.int32).reshape(N_NODES // 2000, 1, 2000)
    eye8 = jnp.eye(8, 16, dtype=f32)
    eye32 = jnp.eye(32, dtype=f32)
    sel32 = jnp.eye(32, 16, dtype=f32)
    srcp = jnp.pad(src, (0, EPADS - N_EDGES), constant_values=1 << 29)

    sc_gather, sc_scatter = _sc_kernels()
    P, Q, cT = _tc_init(node_feats, c16, ws[0], wd[0], eye8)
    nf = node_feats
    EA = edge_attr
    for l in range(N_LAYERS):
        S, D = sc_gather(P, Q, cT, src, dst)
        et = _tc_edge(S, D, EA, we[l], b1[l], wr[l], w2[l], b2[l],
                      cw1[l], cb1[l], cw2[l], sel32, eye32)
        agg = sc_scatter(et, srcp).reshape(NSH, NPAD, 32)
        if l < N_LAYERS - 1:
            nf, P, Q, c16, cT = _tc_node_mid(
                nf, agg, c16, wcm[l], wcmb[l], wnf[l], wne[l],
                wnc[l], nb1[l], wn2[l], nb2[l], ws[l + 1], wd[l + 1], eye8)
            EA = et
        else:
            node_embs, c16f, graph_emb = _tc_node_final(
                nf, agg, c16, batch3, wcm[l], wcmb[l], wnf[l],
                wne[l], wnc[l], nb1[l], wn2[l], nb2[l], pw, pb)

    coords_out = c16f[:, 0:3]
    return (node_embs, graph_emb, coords_out)


# speedup vs baseline: 3.6348x; 1.4519x over previous
import functools

import jax
import jax.numpy as jnp
from jax import lax
from jax.experimental import pallas as pl
from jax.experimental.pallas import tpu as pltpu
from jax.experimental.pallas import tpu_sc as plsc

N_NODES = 10000
N_EDGES = 320000
D_FEAT = 128
D_EDGE = 16
H = 128
N_GRAPHS = 64
N_LAYERS = 3

NC = 2
NS = 16
NW = NC * NS
EPW = N_EDGES // NW
CH = 80
NCH = EPW // CH
NPAD = 10240
NSH = 8
NPART = 4
PROWS = NPAD // NPART
CHS = 256
ESH = 40960
EPADS = NSH * ESH


@functools.cache
def _sc_kernels():
    mesh = plsc.VectorSubcoreMesh(core_axis_name="c", subcore_axis_name="s",
                                  num_cores=NC, num_subcores=NS)

    @functools.partial(
        pl.kernel,
        out_type=(
            jax.ShapeDtypeStruct((N_EDGES, D_FEAT), jnp.float32),
            jax.ShapeDtypeStruct((N_EDGES, 16), jnp.float32),
        ),
        mesh=mesh,
        compiler_params=pltpu.CompilerParams(needs_layout_passes=False),
        scratch_types=[
            pltpu.VMEM((CH,), jnp.int32),
            pltpu.VMEM((CH,), jnp.int32),
            pltpu.VMEM((CH,), jnp.int32),
            pltpu.VMEM((CH,), jnp.int32),
            pltpu.VMEM((CH, D_FEAT), jnp.float32),
            pltpu.VMEM((CH, D_FEAT), jnp.float32),
            pltpu.VMEM((CH, D_FEAT), jnp.float32),
            pltpu.VMEM((CH, D_FEAT), jnp.float32),
            pltpu.VMEM((CH, 16), jnp.float32),
            pltpu.VMEM((CH, 16), jnp.float32),
            pltpu.VMEM((N_NODES // 2000, 4, 2000), jnp.float32),
            pltpu.SemaphoreType.DMA,
            pltpu.SemaphoreType.DMA,
            pltpu.SemaphoreType.DMA,
        ],
    )
    def _sc_gather(p_hbm, q_hbm, ct_hbm, src_hbm, dst_hbm,
                   s_out, d_out, ixs0, ixd0, ixs1, ixd1, bp0, bq0, bp1, bq1,
                   bc0, bc1, xyz, sem_i, sem_g, sem_w):
        wid = lax.axis_index("s") * NC + lax.axis_index("c")
        base = wid * EPW
        IDX = [(ixs0, ixd0), (ixs1, ixd1)]
        BUF = [(bp0, bq0, bc0), (bp1, bq1, bc1)]

        pltpu.sync_copy(ct_hbm, xyz)

        def zrow(j, carry):
            bc0[j, pl.ds(0, 16)] = jnp.zeros((16,), jnp.float32)
            bc1[j, pl.ds(0, 16)] = jnp.zeros((16,), jnp.float32)
            return carry

        lax.fori_loop(0, CH, zrow, 0, unroll=4)

        def issue_idx(i, s):
            off = base + i * CH
            pltpu.async_copy(src_hbm.at[pl.ds(off, CH)], IDX[s][0], sem_i)
            pltpu.async_copy(dst_hbm.at[pl.ds(off, CH)], IDX[s][1], sem_i)

        def wait_idx(s):
            pltpu.make_async_copy(src_hbm.at[pl.ds(0, CH)], IDX[s][0],
                                  sem_i).wait()
            pltpu.make_async_copy(src_hbm.at[pl.ds(0, CH)], IDX[s][1],
                                  sem_i).wait()

        def issue_g(s):
            pltpu.async_copy(p_hbm.at[IDX[s][0]], BUF[s][0], sem_g)
            pltpu.async_copy(q_hbm.at[IDX[s][1]], BUF[s][1], sem_g)

        def wait_g(s):
            pltpu.make_async_copy(p_hbm.at[IDX[s][0]], BUF[s][0],
                                  sem_g).wait()
            pltpu.make_async_copy(p_hbm.at[IDX[s][0]], BUF[s][1],
                                  sem_g).wait()

        def issue_w(i, s):
            off = base + i * CH
            pltpu.async_copy(BUF[s][0], s_out.at[pl.ds(off, CH)], sem_w)
            pltpu.async_copy(BUF[s][2], d_out.at[pl.ds(off, CH)], sem_w)

        def wait_w(s):
            pltpu.make_async_copy(BUF[s][0], s_out.at[pl.ds(0, CH)],
                                  sem_w).wait()
            pltpu.make_async_copy(BUF[s][2], d_out.at[pl.ds(0, CH)],
                                  sem_w).wait()

        def compute(s):
            idx_s, idx_d = IDX[s]
            buf_p, buf_q, buf_c = BUF[s]

            def group(g, c2):
                s16 = idx_s[pl.ds(g * 16, 16)]
                d16 = idx_d[pl.ds(g * 16, 16)]
                sb = s16 // 2000
                sc = s16 - sb * 2000
                db = d16 // 2000
                dc_ = d16 - db * 2000
                rows = lax.iota(jnp.int32, 16) + g * 16
                for c in range(3):
                    csplat = jnp.full((16,), c, jnp.int32)
                    dcv = (plsc.load_gather(xyz, [sb, csplat, sc])
                           - plsc.load_gather(xyz, [db, csplat, dc_]))
                    plsc.store_scatter(buf_c, [rows, jnp.full((16,), c,
                                                              jnp.int32)],
                                       dcv)
                return c2

            def edge(j, c2):
                for k in range(D_FEAT // 16):
                    plsc.addupdate(buf_p.at[j, pl.ds(k * 16, 16)],
                                   buf_q[j, pl.ds(k * 16, 16)])
                return c2

            lax.fori_loop(0, CH // 16, group, 0)
            lax.fori_loop(0, CH, edge, 0, unroll=2)

        issue_idx(0, 0)
        wait_idx(0)
        issue_g(0)
        issue_idx(1, 1)

        def stage(i, s):
            @pl.when(i + 1 < NCH)
            def _():
                wait_idx(1 - s)

                @pl.when(i >= 1)
                def _():
                    wait_w(1 - s)

                issue_g(1 - s)

            wait_g(s)
            compute(s)

            @pl.when(i + 2 < NCH)
            def _():
                issue_idx(i + 2, s)

            issue_w(i, s)

        def chunk(i, carry):
            @pl.when(i % 2 == 0)
            def _():
                stage(i, 0)

            @pl.when(i % 2 == 1)
            def _():
                stage(i, 1)

            return carry

        lax.fori_loop(0, NCH, chunk, 0)
        wait_w((NCH - 2) % 2)
        wait_w((NCH - 1) % 2)

    @functools.partial(
        pl.kernel,
        out_type=jax.ShapeDtypeStruct((NSH * NPAD * 32,), jnp.float32),
        mesh=mesh,
        compiler_params=pltpu.CompilerParams(needs_layout_passes=False),
        scratch_types=[
            pltpu.VMEM((CHS,), jnp.int32),
            pltpu.VMEM((CHS,), jnp.int32),
            pltpu.VMEM((32, CHS), jnp.float32),
            pltpu.VMEM((32, CHS), jnp.float32),
            pltpu.VMEM((PROWS * 32,), jnp.float32),
            pltpu.SemaphoreType.DMA,
        ],
    )
    def _sc_scatter(ett_hbm, src_hbm, agg_out, ix0, ix1, bt0, bt1, tflat,
                    sem_l):
        wid = lax.axis_index("s") * NC + lax.axis_index("c")
        share = wid // NPART
        part = wid - share * NPART
        ebase = share * ESH
        lo = part * PROWS
        NCS = ESH // CHS
        IDX = [ix0, ix1]
        BT = [bt0, bt1]

        zvec = jnp.zeros((16,), jnp.float32)

        def zrow(i, carry):
            tflat[pl.ds(i * 16, 16)] = zvec
            return carry

        lax.fori_loop(0, PROWS * 2, zrow, 0, unroll=4)

        def issue_l(i, s):
            off = ebase + i * CHS
            pltpu.async_copy(src_hbm.at[pl.ds(off, CHS)], IDX[s], sem_l)
            pltpu.async_copy(ett_hbm.at[:, pl.ds(off, CHS)], BT[s], sem_l)

        def wait_l(s):
            pltpu.make_async_copy(src_hbm.at[pl.ds(0, CHS)], IDX[s],
                                  sem_l).wait()
            pltpu.make_async_copy(ett_hbm.at[:, pl.ds(0, CHS)], BT[s],
                                  sem_l).wait()

        def compute(s):
            idx_v = IDX[s]
            buf_t = BT[s]

            def group(g, c2):
                s16 = idx_v[pl.ds(g * 16, 16)]
                tgt = s16 - lo
                mask = (tgt >= 0) & (tgt < PROWS)
                tgt = jnp.minimum(jnp.maximum(tgt, 0), PROWS - 1)
                fidx = tgt * 32
                for c in range(32):
                    vec = buf_t[c, pl.ds(g * 16, 16)]
                    plsc.addupdate_scatter(tflat, [fidx + c], vec, mask=mask)
                return c2

            lax.fori_loop(0, CHS // 16, group, 0)

        issue_l(0, 0)

        def stage(i, s):
            @pl.when(i + 1 < NCS)
            def _():
                issue_l(i + 1, 1 - s)

            wait_l(s)
            compute(s)

        def chunk(i, carry):
            @pl.when(i % 2 == 0)
            def _():
                stage(i, 0)

            @pl.when(i % 2 == 1)
            def _():
                stage(i, 1)

            return carry

        lax.fori_loop(0, NCS, chunk, 0)
        pltpu.sync_copy(
            tflat,
            agg_out.at[pl.ds(share * (NPAD * 32) + part * (PROWS * 32),
                             PROWS * 32)])

    return _sc_gather, _sc_scatter


_NT = (((1,), (1,)), ((), ()))


def _init_body(nf_ref, c16_ref, ws_ref, wd_ref, eye_ref, p_ref, q_ref,
               ct_ref):
    nf = nf_ref[...]
    p_ref[...] = jnp.dot(nf, ws_ref[...], preferred_element_type=jnp.float32)
    q_ref[...] = jnp.dot(nf, wd_ref[...], preferred_element_type=jnp.float32)
    ct_ref[0] = lax.dot_general(eye_ref[...], c16_ref[...], _NT,
                                preferred_element_type=jnp.float32)


def _edge_body(s_ref, d_ref, ea_ref, we_ref, b1_ref, wr_ref, w2_ref, b2_ref,
               cw1_ref, cb1_ref, cw2_ref, sel_ref, eye_ref, et_ref, *,
               ea_rows):
    S = s_ref[...]
    D = d_ref[...]
    if ea_rows:
        EA = lax.dot_general(ea_ref[...], sel_ref[...],
                             (((0,), (0,)), ((), ())),
                             preferred_element_type=jnp.float32)
    else:
        EA = ea_ref[...]
    radial = jnp.sum(D * D, axis=1, keepdims=True)
    pre = S + jnp.dot(EA, we_ref[...], preferred_element_type=jnp.float32)
    pre = pre + radial * wr_ref[...] + b1_ref[...]
    h = jnp.maximum(pre, 0.0)
    eo = jnp.dot(h, w2_ref[...], preferred_element_type=jnp.float32) + b2_ref[...]
    t0 = jnp.maximum(
        jnp.sum(eo * cw1_ref[0:1, :], axis=1, keepdims=True) + cb1_ref[0, 0], 0.0)
    t1 = jnp.maximum(
        jnp.sum(eo * cw1_ref[1:2, :], axis=1, keepdims=True) + cb1_ref[0, 1], 0.0)
    eps = t0 * cw2_ref[0, 0] + t1 * cw2_ref[0, 1]
    norm = jnp.sqrt(radial) + 1.0
    trans = (D / norm) * eps
    cat = jnp.concatenate([eo, trans], axis=1)
    et_ref[...] = lax.dot_general(eye_ref[...], cat, _NT,
                                  preferred_element_type=jnp.float32)


def _node_mid_body(nf_ref, agg_ref, c16_ref,
                   wcm_ref, wcmb_ref, wnf_ref, wne_ref, wnc_ref, nb1_ref,
                   wn2_ref, nb2_ref, wsn_ref, wdn_ref, eye_ref,
                   nf2_ref, p_ref, q_ref, c16o_ref, ct_ref):
    agg = jnp.sum(agg_ref[...], axis=0)
    agg_e = agg[:, 0:16]
    agg_c = agg[:, 16:32]
    acc = c16_ref[...] + agg_c
    cout = jnp.dot(acc, wcm_ref[...], preferred_element_type=jnp.float32)
    cout = cout + wcmb_ref[...]
    hn = jnp.dot(nf_ref[...], wnf_ref[...], preferred_element_type=jnp.float32)
    hn = hn + jnp.dot(agg_e, wne_ref[...], preferred_element_type=jnp.float32)
    hn = hn + jnp.dot(cout, wnc_ref[...], preferred_element_type=jnp.float32)
    hn = jnp.maximum(hn + nb1_ref[...], 0.0)
    nf2 = jnp.dot(hn, wn2_ref[...], preferred_element_type=jnp.float32) + nb2_ref[...]
    nf2_ref[...] = nf2
    p_ref[...] = jnp.dot(nf2, wsn_ref[...], preferred_element_type=jnp.float32)
    q_ref[...] = jnp.dot(nf2, wdn_ref[...], preferred_element_type=jnp.float32)
    c16o_ref[...] = cout
    ct_ref[0] = lax.dot_general(eye_ref[...], cout, _NT,
                                preferred_element_type=jnp.float32)


def _node_final_body(nf_ref, agg_ref, c16_ref, batch_ref,
                     wcm_ref, wcmb_ref, wnf_ref, wne_ref, wnc_ref, nb1_ref,
                     wn2_ref, nb2_ref, pw_ref, pb_ref,
                     emb_ref, c16o_ref, graph_ref, sums_ref, counts_ref,
                     *, n_steps, block_n):
    step = pl.program_id(0)
    agg = jnp.sum(agg_ref[...], axis=0)
    agg_e = agg[:, 0:16]
    agg_c = agg[:, 16:32]
    acc = c16_ref[...] + agg_c
    cout = jnp.dot(acc, wcm_ref[...], preferred_element_type=jnp.float32)
    cout = cout + wcmb_ref[...]
    hn = jnp.dot(nf_ref[...], wnf_ref[...], preferred_element_type=jnp.float32)
    hn = hn + jnp.dot(agg_e, wne_ref[...], preferred_element_type=jnp.float32)
    hn = hn + jnp.dot(cout, wnc_ref[...], preferred_element_type=jnp.float32)
    hn = jnp.maximum(hn + nb1_ref[...], 0.0)
    nf2 = jnp.dot(hn, wn2_ref[...], preferred_element_type=jnp.float32) + nb2_ref[...]
    embs = jnp.dot(nf2, pw_ref[...], preferred_element_type=jnp.float32) + pb_ref[...]
    emb_ref[...] = embs
    c16o_ref[...] = cout

    bvec = batch_ref[0]
    oh = (lax.broadcasted_iota(jnp.int32, (N_GRAPHS, block_n), 0)
          == bvec).astype(jnp.float32)
    s_step = jnp.dot(oh, embs, preferred_element_type=jnp.float32)
    c_step = jnp.sum(oh, axis=1, keepdims=True)

    @pl.when(step == 0)
    def _():
        sums_ref[...] = jnp.zeros_like(sums_ref)
        counts_ref[...] = jnp.zeros_like(counts_ref)

    sums_ref[...] += s_step
    counts_ref[...] += c_step

    @pl.when(step == n_steps - 1)
    def _():
        graph_ref[...] = sums_ref[...] / jnp.maximum(counts_ref[...], 1.0)


def _bspec(block, imap):
    return pl.BlockSpec(block, imap)


def _full(shape):
    return pl.BlockSpec(shape, lambda i: tuple(0 for _ in shape))


def _tc_init(nf, c16, ws0, wd0, eye8, block_n=2000):
    n_steps = N_NODES // block_n
    return pl.pallas_call(
        _init_body,
        grid=(n_steps,),
        in_specs=[
            _bspec((block_n, D_FEAT), lambda i: (i, 0)),
            _bspec((block_n, 16), lambda i: (i, 0)),
            _full((D_FEAT, D_FEAT)),
            _full((D_FEAT, D_FEAT)),
            _full((4, 16)),
        ],
        out_specs=[
            _bspec((block_n, D_FEAT), lambda i: (i, 0)),
            _bspec((block_n, D_FEAT), lambda i: (i, 0)),
            _bspec((1, 4, block_n), lambda i: (i, 0, 0)),
        ],
        out_shape=[
            jax.ShapeDtypeStruct((N_NODES, D_FEAT), jnp.float32),
            jax.ShapeDtypeStruct((N_NODES, D_FEAT), jnp.float32),
            jax.ShapeDtypeStruct((n_steps, 4, block_n), jnp.float32),
        ],
    )(nf, c16, ws0, wd0, eye8)


def _tc_edge(S, D, EA, we, b1, wr, w2, b2, cw1, cb1, cw2, sel, eye32,
             block_e=2560):
    n_steps = N_EDGES // block_e
    ea_rows = (EA.shape[0] == 32)
    ea_spec = (_bspec((32, block_e), lambda i: (0, i)) if ea_rows
               else _bspec((block_e, D_EDGE), lambda i: (i, 0)))
    body = functools.partial(_edge_body, ea_rows=ea_rows)
    return pl.pallas_call(
        body,
        grid=(n_steps,),
        in_specs=[
            _bspec((block_e, D_FEAT), lambda i: (i, 0)),
            _bspec((block_e, 16), lambda i: (i, 0)),
            ea_spec,
            _full((D_EDGE, H)),
            _full((1, H)),
            _full((1, H)),
            _full((H, D_EDGE)),
            _full((1, D_EDGE)),
            _full((2, D_EDGE)),
            _full((1, 2)),
            _full((1, 2)),
            _full((32, D_EDGE)),
            _full((32, 32)),
        ],
        out_specs=[_bspec((32, block_e), lambda i: (0, i))],
        out_shape=[jax.ShapeDtypeStruct((32, EPADS), jnp.float32)],
    )(S, D, EA, we, b1, wr, w2, b2, cw1, cb1, cw2, sel, eye32)[0]


def _tc_node_mid(nf, agg, c16, wcm, wcmb, wnf, wne, wnc, nb1, wn2, nb2,
                 wsn, wdn, eye8, block_n=2000):
    n_steps = N_NODES // block_n
    row = lambda i: (i, 0)
    return pl.pallas_call(
        _node_mid_body,
        grid=(n_steps,),
        in_specs=[
            _bspec((block_n, D_FEAT), row),
            _bspec((NSH, block_n, 32), lambda i: (0, i, 0)),
            _bspec((block_n, 16), row),
            _full((16, 16)),
            _full((1, 16)),
            _full((D_FEAT, H)),
            _full((16, H)),
            _full((16, H)),
            _full((1, H)),
            _full((H, D_FEAT)),
            _full((1, D_FEAT)),
            _full((D_FEAT, D_FEAT)),
            _full((D_FEAT, D_FEAT)),
            _full((4, 16)),
        ],
        out_specs=[
            _bspec((block_n, D_FEAT), row),
            _bspec((block_n, D_FEAT), row),
            _bspec((block_n, D_FEAT), row),
            _bspec((block_n, 16), row),
            _bspec((1, 4, block_n), lambda i: (i, 0, 0)),
        ],
        out_shape=[
            jax.ShapeDtypeStruct((N_NODES, D_FEAT), jnp.float32),
            jax.ShapeDtypeStruct((N_NODES, D_FEAT), jnp.float32),
            jax.ShapeDtypeStruct((N_NODES, D_FEAT), jnp.float32),
            jax.ShapeDtypeStruct((NPAD, 16), jnp.float32),
            jax.ShapeDtypeStruct((n_steps, 4, block_n), jnp.float32),
        ],
    )(nf, agg, c16, wcm, wcmb, wnf, wne, wnc, nb1, wn2, nb2, wsn, wdn,
      eye8)


def _tc_node_final(nf, agg, c16, batch3, wcm, wcmb, wnf, wne, wnc, nb1,
                   wn2, nb2, pw, pb, block_n=2000):
    n_steps = N_NODES // block_n
    row = lambda i: (i, 0)
    body = functools.partial(_node_final_body, n_steps=n_steps, block_n=block_n)
    return pl.pallas_call(
        body,
        grid=(n_steps,),
        in_specs=[
            _bspec((block_n, D_FEAT), row),
            _bspec((NSH, block_n, 32), lambda i: (0, i, 0)),
            _bspec((block_n, 16), row),
            _bspec((1, 1, block_n), lambda i: (i, 0, 0)),
            _full((16, 16)),
            _full((1, 16)),
            _full((D_FEAT, H)),
            _full((16, H)),
            _full((16, H)),
            _full((1, H)),
            _full((H, D_FEAT)),
            _full((1, D_FEAT)),
            _full((D_FEAT, D_FEAT)),
            _full((1, D_FEAT)),
        ],
        out_specs=[
            _bspec((block_n, D_FEAT), row),
            _bspec((block_n, 16), row),
            _full((N_GRAPHS, D_FEAT)),
        ],
        out_shape=[
            jax.ShapeDtypeStruct((N_NODES, D_FEAT), jnp.float32),
            jax.ShapeDtypeStruct((N_NODES, 16), jnp.float32),
            jax.ShapeDtypeStruct((N_GRAPHS, D_FEAT), jnp.float32),
        ],
        scratch_shapes=[
            pltpu.VMEM((N_GRAPHS, D_FEAT), jnp.float32),
            pltpu.VMEM((N_GRAPHS, 1), jnp.float32),
        ],
    )(nf, agg, c16, batch3, wcm, wcmb, wnf, wne, wnc, nb1, wn2, nb2,
      pw, pb)


def kernel(node_feats, edge_index, edge_attr, coords, batch_node_vec,
           ew1s, eb1s, ew2s, eb2s, nw1s, nb1s, nw2s, nb2s,
           cw1s, cb1s, cw2s, cmws, cmbs, post_w, post_b):
    f32 = jnp.float32
    src = edge_index[0]
    dst = edge_index[1]

    ws = [ew1s[l][:, 0:D_FEAT].T for l in range(N_LAYERS)]
    wd = [ew1s[l][:, D_FEAT:2 * D_FEAT].T for l in range(N_LAYERS)]
    we = [ew1s[l][:, 2 * D_FEAT:2 * D_FEAT + D_EDGE].T for l in range(N_LAYERS)]
    wr = [ew1s[l][:, 2 * D_FEAT + D_EDGE:].T for l in range(N_LAYERS)]
    b1 = [eb1s[l].reshape(1, H) for l in range(N_LAYERS)]
    w2 = [ew2s[l].T for l in range(N_LAYERS)]
    b2 = [eb2s[l].reshape(1, D_EDGE) for l in range(N_LAYERS)]
    cw1 = [cw1s[l] for l in range(N_LAYERS)]
    cb1 = [cb1s[l].reshape(1, 2) for l in range(N_LAYERS)]
    cw2 = [cw2s[l] for l in range(N_LAYERS)]
    wcm = [jnp.zeros((16, 16), f32).at[0:3, 0:3].set(cmws[l].T)
           for l in range(N_LAYERS)]
    wcmb = [jnp.zeros((1, 16), f32).at[0, 0:3].set(cmbs[l])
            for l in range(N_LAYERS)]
    wnf = [nw1s[l][:, 0:D_FEAT].T for l in range(N_LAYERS)]
    wne = [nw1s[l][:, D_FEAT:D_FEAT + D_EDGE].T for l in range(N_LAYERS)]
    wnc = [jnp.zeros((16, H), f32).at[0:3, :].set(nw1s[l][:, D_FEAT + D_EDGE:].T)
           for l in range(N_LAYERS)]
    nb1 = [nb1s[l].reshape(1, H) for l in range(N_LAYERS)]
    wn2 = [nw2s[l].T for l in range(N_LAYERS)]
    nb2 = [nb2s[l].reshape(1, D_FEAT) for l in range(N_LAYERS)]
    pw = post_w.T
    pb = post_b.reshape(1, D_FEAT)

    c16 = jnp.pad(coords, ((0, NPAD - N_NODES), (0, 13)))
    batch3 = batch_node_vec.astype(jnp.int32).reshape(N_NODES // 2000, 1, 2000)
    eye8 = jnp.eye(4, 16, dtype=f32)
    eye32 = jnp.eye(32, dtype=f32)
    sel32 = jnp.eye(32, 16, dtype=f32)
    srcp = jnp.pad(src, (0, EPADS - N_EDGES), constant_values=1 << 29)

    sc_gather, sc_scatter = _sc_kernels()
    P, Q, cT = _tc_init(node_feats, c16, ws[0], wd[0], eye8)
    nf = node_feats
    EA = edge_attr
    for l in range(N_LAYERS):
        S, D = sc_gather(P, Q, cT, src, dst)
        et = _tc_edge(S, D, EA, we[l], b1[l], wr[l], w2[l], b2[l],
                      cw1[l], cb1[l], cw2[l], sel32, eye32)
        agg = sc_scatter(et, srcp).reshape(NSH, NPAD, 32)
        if l < N_LAYERS - 1:
            nf, P, Q, c16, cT = _tc_node_mid(
                nf, agg, c16, wcm[l], wcmb[l], wnf[l], wne[l],
                wnc[l], nb1[l], wn2[l], nb2[l], ws[l + 1], wd[l + 1], eye8)
            EA = et
        else:
            node_embs, c16f, graph_emb = _tc_node_final(
                nf, agg, c16, batch3, wcm[l], wcmb[l], wnf[l],
                wne[l], wnc[l], nb1[l], wn2[l], nb2[l], pw, pb)

    coords_out = c16f[:, 0:3]
    return (node_embs, graph_emb, coords_out)
